# baseline (speedup 1.0000x reference)
import functools

import jax
import jax.numpy as jnp
from jax.experimental import pallas as pl
from jax.experimental.pallas import tpu as pltpu

_PROJ_DIM = 128
_CLASS_NUM = 7
_QUEUE_SIZE = 168
_TEMPERATURE = 0.5
_QC = _QUEUE_SIZE * _CLASS_NUM

_BLOCK_B = 1024
_LBL_ROWS = _BLOCK_B // 128
_LOG2E = 1.4426950408889634
_LN2 = 0.6931471805599453


def _pgc_kernel(act_ref, ema_ref, lbl_ref, queue_ref, out_ref, qbf_ref, qb_ref):
    i = pl.program_id(0)

    @pl.when(i == 0)
    def _prep():
        queue_bf = queue_ref[...].astype(jnp.bfloat16)
        qbf_ref[...] = queue_bf
        col_of = jax.lax.broadcasted_iota(jnp.int32, (8, _QC), 1) // _QUEUE_SIZE
        row_of = jax.lax.broadcasted_iota(jnp.int32, (8, _QC), 0)
        indicator = jnp.where(col_of == row_of, 1.0, 0.0).astype(jnp.bfloat16)
        qb_ref[...] = jax.lax.dot_general(
            indicator, queue_bf,
            dimension_numbers=(((1,), (0,)), ((), ())),
            preferred_element_type=jnp.float32,
        )

    act = act_ref[...]
    ema = ema_ref[...]

    ssq_a = jnp.sum(act * act, axis=1, keepdims=True)
    ssq_e = jnp.sum(ema * ema, axis=1, keepdims=True)
    rn_a = jax.lax.rsqrt(jnp.maximum(ssq_a, 1e-24))
    rn_e = jax.lax.rsqrt(jnp.maximum(ssq_e, 1e-24))

    l_pos = jnp.sum(act * ema, axis=1, keepdims=True) * (rn_a * rn_e)

    fs = (act * (rn_a * (_LOG2E / _TEMPERATURE))).astype(jnp.bfloat16)

    sim_s = jax.lax.dot_general(
        fs, qbf_ref[...],
        dimension_numbers=(((1,), (1,)), ((), ())),
        preferred_element_type=jnp.float32,
    )

    total = jnp.sum(jnp.exp2(sim_s), axis=1, keepdims=True)
    denom = l_pos + total

    lbl = lbl_ref[...]
    subl = jax.lax.broadcasted_iota(jnp.int32, (8, 128), 0)
    chunks = [
        jnp.where(jnp.broadcast_to(lbl[k : k + 1, :], (8, 128)) == subl, 1.0, 0.0)
        for k in range(_LBL_ROWS)
    ]
    onehot_t = jnp.concatenate(chunks, axis=1).astype(jnp.bfloat16)

    g = jax.lax.dot_general(
        onehot_t, fs,
        dimension_numbers=(((1,), (0,)), ((), ())),
        preferred_element_type=jnp.float32,
    )
    masked_total_s = jnp.sum(g * qb_ref[...], axis=(0, 1), keepdims=True)

    d_t = jnp.transpose(denom)
    lp_t = jnp.transpose(l_pos + 1e-8 * denom)
    rows_total = (
        (_QUEUE_SIZE + 1) * jnp.sum(jnp.log(d_t), axis=1, keepdims=True)
        - jnp.sum(jnp.log(lp_t), axis=1, keepdims=True)
    )

    scale = 1.0 / ((_QUEUE_SIZE + 1) * _BLOCK_B * pl.num_programs(0))
    partial = (rows_total - masked_total_s * _LN2) * scale

    @pl.when(i == 0)
    def _init():
        out_ref[...] = jnp.zeros_like(out_ref)

    out_ref[...] += partial


@functools.partial(jax.jit, static_argnames=())
def kernel(activation, ema_activation, pseudo_label, queue_list):
    batch = activation.shape[0]
    labels = pseudo_label.reshape(batch // 128, 128)
    grid = (batch // _BLOCK_B,)

    out = pl.pallas_call(
        _pgc_kernel,
        grid=grid,
        in_specs=[
            pl.BlockSpec((_BLOCK_B, _PROJ_DIM), lambda i: (i, 0)),
            pl.BlockSpec((_BLOCK_B, _PROJ_DIM), lambda i: (i, 0)),
            pl.BlockSpec((_LBL_ROWS, 128), lambda i: (i, 0)),
            pl.BlockSpec((_QC, _PROJ_DIM), lambda i: (0, 0)),
        ],
        out_specs=pl.BlockSpec((1, 1), lambda i: (0, 0)),
        out_shape=jax.ShapeDtypeStruct((1, 1), jnp.float32),
        scratch_shapes=[
            pltpu.VMEM((_QC, _PROJ_DIM), jnp.bfloat16),
            pltpu.VMEM((8, _PROJ_DIM), jnp.float32),
        ],
    )(activation, ema_activation, labels, queue_list)

    return out[0, 0]


# speedup vs baseline: 1.2678x; 1.2678x over previous
import functools

import jax
import jax.numpy as jnp
from jax.experimental import pallas as pl
from jax.experimental.pallas import tpu as pltpu

_PROJ_DIM = 128
_CLASS_NUM = 7
_QUEUE_SIZE = 168
_TEMPERATURE = 0.5
_QC = _QUEUE_SIZE * _CLASS_NUM

_BLOCK_B = 1024
_LBL_ROWS = _BLOCK_B // 128
_LOG2E = 1.4426950408889634
_LN2 = 0.6931471805599453


def _pgc_kernel(act_ref, ema_ref, lbl_ref, queue_ref, out_ref, qbf_ref, qb_ref):
    i = pl.program_id(0)

    @pl.when(i == 0)
    def _prep():
        queue_bf = queue_ref[...].astype(jnp.bfloat16)
        qbf_ref[...] = queue_bf
        col_of = jax.lax.broadcasted_iota(jnp.int32, (8, _QC), 1) // _QUEUE_SIZE
        row_of = jax.lax.broadcasted_iota(jnp.int32, (8, _QC), 0)
        indicator = jnp.where(col_of == row_of, 1.0, 0.0).astype(jnp.bfloat16)
        qb_ref[...] = jax.lax.dot_general(
            indicator, queue_bf,
            dimension_numbers=(((1,), (0,)), ((), ())),
            preferred_element_type=jnp.float32,
        )

    act = act_ref[...]
    ema = ema_ref[...]

    ssq_a = jnp.sum(act * act, axis=1, keepdims=True)
    ssq_e = jnp.sum(ema * ema, axis=1, keepdims=True)
    rn_a = jax.lax.rsqrt(jnp.maximum(ssq_a, 1e-24))
    rn_e = jax.lax.rsqrt(jnp.maximum(ssq_e, 1e-24))

    l_pos = jnp.sum(act * ema, axis=1, keepdims=True) * (rn_a * rn_e)

    fs = (act * (rn_a * (_LOG2E / _TEMPERATURE))).astype(jnp.bfloat16)

    sim_s = jax.lax.dot_general(
        fs, qbf_ref[...],
        dimension_numbers=(((1,), (1,)), ((), ())),
        preferred_element_type=jnp.float32,
    )

    total = jnp.sum(jnp.exp2(sim_s), axis=1, keepdims=True)
    denom = l_pos + total

    lbl = lbl_ref[...]
    subl = jax.lax.broadcasted_iota(jnp.int32, (8, 128), 0)
    chunks = [
        jnp.where(jnp.broadcast_to(lbl[k : k + 1, :], (8, 128)) == subl, 1.0, 0.0)
        for k in range(_LBL_ROWS)
    ]
    onehot_t = jnp.concatenate(chunks, axis=1).astype(jnp.bfloat16)

    g = jax.lax.dot_general(
        onehot_t, fs,
        dimension_numbers=(((1,), (0,)), ((), ())),
        preferred_element_type=jnp.float32,
    )
    masked_total_s = jnp.sum(g * qb_ref[...], axis=(0, 1), keepdims=True)

    rows_total = jnp.sum(
        (_QUEUE_SIZE + 1) * jnp.log(denom) - jnp.log(l_pos + 1e-8 * denom),
        axis=0,
        keepdims=True,
    )[:, :1]

    scale = 1.0 / ((_QUEUE_SIZE + 1) * _BLOCK_B * pl.num_programs(0))
    partial = (rows_total - masked_total_s * _LN2) * scale

    @pl.when(i == 0)
    def _init():
        out_ref[...] = jnp.zeros_like(out_ref)

    out_ref[...] += partial


@functools.partial(jax.jit, static_argnames=())
def kernel(activation, ema_activation, pseudo_label, queue_list):
    batch = activation.shape[0]
    labels = pseudo_label.reshape(batch // 128, 128)
    grid = (batch // _BLOCK_B,)

    out = pl.pallas_call(
        _pgc_kernel,
        grid=grid,
        in_specs=[
            pl.BlockSpec((_BLOCK_B, _PROJ_DIM), lambda i: (i, 0)),
            pl.BlockSpec((_BLOCK_B, _PROJ_DIM), lambda i: (i, 0)),
            pl.BlockSpec((_LBL_ROWS, 128), lambda i: (i, 0)),
            pl.BlockSpec((_QC, _PROJ_DIM), lambda i: (0, 0)),
        ],
        out_specs=pl.BlockSpec((1, 1), lambda i: (0, 0)),
        out_shape=jax.ShapeDtypeStruct((1, 1), jnp.float32),
        scratch_shapes=[
            pltpu.VMEM((_QC, _PROJ_DIM), jnp.bfloat16),
            pltpu.VMEM((8, _PROJ_DIM), jnp.float32),
        ],
    )(activation, ema_activation, labels, queue_list)

    return out[0, 0]
